# baseline (speedup 1.0000x reference)
import functools

import jax
import jax.numpy as jnp
from jax import lax
from jax.experimental import pallas as pl
from jax.experimental.pallas import tpu as pltpu
from jax.experimental.pallas import tpu_sc as plsc

VOCAB = 1000000
EMBED = 64
SEQ = 50
BATCH = 4096

NC = 2
NS = 16
NW = NC * NS

TOKENS = BATCH * SEQ
PER_W = TOKENS // NW
CHUNK = 400
SEQ_PER_CHUNK = CHUNK // SEQ
N_CHUNKS = PER_W // CHUNK
SUBG = 80
N_SUBG = CHUNK // SUBG
RING = 4
N_PAIR = N_CHUNKS // RING


def _make_sc_kernel():
    mesh = plsc.VectorSubcoreMesh(core_axis_name="c", subcore_axis_name="s")

    @functools.partial(
        pl.kernel,
        mesh=mesh,
        out_type=jax.ShapeDtypeStruct((TOKENS, EMBED), jnp.float32),
        compiler_params=pltpu.CompilerParams(use_tc_tiling_on_sc=False),
        scratch_types=[
            pltpu.VMEM((RING, N_SUBG, SUBG), jnp.int32),
            pltpu.VMEM((RING, SEQ_PER_CHUNK, 64), jnp.float32),
            pltpu.VMEM((RING, CHUNK, EMBED), jnp.float32),
            pltpu.VMEM((SEQ, EMBED), jnp.float32),
        ] + [pltpu.SemaphoreType.DMA] * (2 * RING),
    )
    def sc_kernel(tok_hbm, maskp_hbm, table_hbm, pos_hbm, out_hbm,
                  idx_v, maskp_v, rows_v, pos_v, *sems):
        gsem = sems[:RING]
        wsem = sems[RING:]
        wid = lax.axis_index("s") * NC + lax.axis_index("c")
        base = wid * PER_W

        pltpu.sync_copy(pos_hbm, pos_v)

        lanes = lax.broadcasted_iota(jnp.int32, (16,), 0)
        dnums = lax.GatherDimensionNumbers(
            offset_dims=(), collapsed_slice_dims=(0,), start_index_map=(0,))

        def cbase_of(ci):
            return pl.multiple_of(base + ci * CHUNK, CHUNK)

        def stage_and_fire(b, ci):
            cbase = cbase_of(ci)
            for g in range(N_SUBG):
                pltpu.sync_copy(
                    tok_hbm.at[pl.ds(cbase + g * SUBG, SUBG)],
                    idx_v.at[b, g])
            seqbase = pl.multiple_of(cbase // SEQ, SEQ_PER_CHUNK)
            pltpu.sync_copy(
                maskp_hbm.at[pl.ds(seqbase, SEQ_PER_CHUNK)], maskp_v.at[b])
            for g in range(N_SUBG):
                pltpu.async_copy(
                    table_hbm.at[idx_v.at[b, g]],
                    rows_v.at[b, pl.ds(g * SUBG, SUBG)],
                    gsem[b])

        def wait_gather(b):
            for g in range(N_SUBG):
                pltpu.make_async_copy(
                    table_hbm.at[idx_v.at[b, g]],
                    rows_v.at[b, pl.ds(g * SUBG, SUBG)],
                    gsem[b]).wait()

        def fire_wb(b, ci):
            pltpu.async_copy(
                rows_v.at[b], out_hbm.at[pl.ds(cbase_of(ci), CHUNK)], wsem[b])

        def wait_wb(b):
            pltpu.make_async_copy(
                rows_v.at[b], out_hbm.at[pl.ds(cbase_of(0), CHUNK)],
                wsem[b]).wait()

        def compute(b):
            def seq_body(si, c2):
                tbase = si * SEQ
                mrow = [maskp_v[b, si, pl.ds(k * 16, 16)] for k in range(4)]
                for s in range(SEQ):
                    t = tbase + s
                    k, lane = divmod(s, 16)
                    idx = lanes * 0 + lane
                    m = lax.gather(
                        mrow[k], idx[:, None], dnums, (1,),
                        mode=lax.GatherScatterMode.PROMISE_IN_BOUNDS)
                    for dg in range(EMBED // 16):
                        sl = pl.ds(dg * 16, 16)
                        rows_v[b, t, sl] = (rows_v[b, t, sl]
                                            + pos_v[s, sl]) * m
                return c2
            lax.fori_loop(0, SEQ_PER_CHUNK, seq_body, 0)

        stage_and_fire(0, 0)
        stage_and_fire(1, 1)

        def pair_body(p, carry):
            for j in range(RING):
                ci = p * RING + j
                fb = (j + 2) % RING
                fci = ci + 2

                @pl.when(jnp.logical_and(fci >= RING, fci < N_CHUNKS))
                def _():
                    wait_wb(fb)

                @pl.when(fci < N_CHUNKS)
                def _():
                    stage_and_fire(fb, fci)

                wait_gather(j)
                compute(j)
                fire_wb(j, ci)
            return carry

        lax.fori_loop(0, N_PAIR, pair_body, 0)
        for b in range(RING):
            wait_wb(b)

    return sc_kernel


_SC_KERNEL = _make_sc_kernel()


@jax.jit
def kernel(tokens, mask, token_embed, pos_embed):
    tok_flat = tokens.reshape(TOKENS).astype(jnp.int32)
    maskp = jnp.pad(mask, ((0, 0), (0, 64 - SEQ)))
    pos = pos_embed.reshape(SEQ, EMBED)
    out = _SC_KERNEL(tok_flat, maskp, token_embed, pos)
    return out.reshape(BATCH, SEQ, EMBED)


# speedup vs baseline: 1.1732x; 1.1732x over previous
import functools

import jax
import jax.numpy as jnp
from jax import lax
from jax.experimental import pallas as pl
from jax.experimental.pallas import tpu as pltpu
from jax.experimental.pallas import tpu_sc as plsc

VOCAB = 1000000
EMBED = 64
SEQ = 50
BATCH = 4096

NC = 2
NS = 16
NW = NC * NS

TOKENS = BATCH * SEQ
PER_W = TOKENS // NW
SEQ_W = PER_W // SEQ
CHUNK = 400
SEQ_PER_CHUNK = CHUNK // SEQ
N_CHUNKS = PER_W // CHUNK
SUBG = 80
N_SUBG = CHUNK // SUBG
IDX_ROWS = PER_W // SUBG
RING = 4
N_OUTER = N_CHUNKS // RING


def _make_sc_kernel():
    mesh = plsc.VectorSubcoreMesh(core_axis_name="c", subcore_axis_name="s")

    @functools.partial(
        pl.kernel,
        mesh=mesh,
        out_type=jax.ShapeDtypeStruct((TOKENS, EMBED), jnp.float32),
        compiler_params=pltpu.CompilerParams(use_tc_tiling_on_sc=False),
        scratch_types=[
            pltpu.VMEM((IDX_ROWS, SUBG), jnp.int32),
            pltpu.VMEM((SEQ_W, 64), jnp.float32),
            pltpu.VMEM((RING, CHUNK, EMBED), jnp.float32),
            pltpu.VMEM((SEQ, EMBED), jnp.float32),
        ] + [pltpu.SemaphoreType.DMA] * (2 * RING),
    )
    def sc_kernel(tok_hbm, maskp_hbm, table_hbm, pos_hbm, out_hbm,
                  idx_v, maskp_v, rows_v, pos_v, *sems):
        gsem = sems[:RING]
        wsem = sems[RING:]
        wid = lax.axis_index("s") * NC + lax.axis_index("c")
        base = wid * PER_W

        pltpu.sync_copy(pos_hbm, pos_v)
        pltpu.sync_copy(
            tok_hbm.at[pl.ds(pl.multiple_of(wid * IDX_ROWS, IDX_ROWS),
                             IDX_ROWS)], idx_v)
        pltpu.sync_copy(
            maskp_hbm.at[pl.ds(pl.multiple_of(wid * SEQ_W, SEQ_W), SEQ_W)],
            maskp_v)

        lanes = lax.broadcasted_iota(jnp.int32, (16,), 0)
        dnums = lax.GatherDimensionNumbers(
            offset_dims=(), collapsed_slice_dims=(0,), start_index_map=(0,))

        def cbase_of(ci):
            return pl.multiple_of(base + ci * CHUNK, CHUNK)

        def fire_gather(b, ci):
            for g in range(N_SUBG):
                pltpu.async_copy(
                    table_hbm.at[idx_v.at[ci * N_SUBG + g]],
                    rows_v.at[b, pl.ds(g * SUBG, SUBG)],
                    gsem[b])

        def wait_gather(b, ci):
            for g in range(N_SUBG):
                pltpu.make_async_copy(
                    table_hbm.at[idx_v.at[ci * N_SUBG + g]],
                    rows_v.at[b, pl.ds(g * SUBG, SUBG)],
                    gsem[b]).wait()

        def fire_wb(b, ci):
            pltpu.async_copy(
                rows_v.at[b], out_hbm.at[pl.ds(cbase_of(ci), CHUNK)], wsem[b])

        def wait_wb(b):
            pltpu.make_async_copy(
                rows_v.at[b], out_hbm.at[pl.ds(cbase_of(0), CHUNK)],
                wsem[b]).wait()

        def compute(b, ci):
            srow0 = ci * SEQ_PER_CHUNK

            def s_body(s, c2):
                k16 = (s // 16) * 16
                lane = s % 16
                gidx = lanes * 0 + lane
                prow = [pos_v[s, pl.ds(dg * 16, 16)] for dg in range(4)]
                for si in range(SEQ_PER_CHUNK):
                    t = si * SEQ + s
                    mrow = maskp_v[srow0 + si, pl.ds(k16, 16)]
                    m = lax.gather(
                        mrow, gidx[:, None], dnums, (1,),
                        mode=lax.GatherScatterMode.PROMISE_IN_BOUNDS)
                    for dg in range(EMBED // 16):
                        sl = pl.ds(dg * 16, 16)
                        rows_v[b, t, sl] = (rows_v[b, t, sl]
                                            + prow[dg]) * m
                return c2

            lax.fori_loop(0, SEQ, s_body, 0)

        fire_gather(0, 0)
        fire_gather(1, 1)

        def outer_body(p, carry):
            for j in range(RING):
                ci = p * RING + j
                fb = (j + 2) % RING
                fci = ci + 2

                @pl.when(jnp.logical_and(fci >= RING, fci < N_CHUNKS))
                def _():
                    wait_wb(fb)

                @pl.when(fci < N_CHUNKS)
                def _():
                    fire_gather(fb, fci)

                wait_gather(j, ci)
                compute(j, ci)
                fire_wb(j, ci)
            return carry

        lax.fori_loop(0, N_OUTER, outer_body, 0)
        for b in range(RING):
            wait_wb(b)

    return sc_kernel


_SC_KERNEL = _make_sc_kernel()


@jax.jit
def kernel(tokens, mask, token_embed, pos_embed):
    tok2d = tokens.reshape(TOKENS // SUBG, SUBG).astype(jnp.int32)
    maskp = jnp.pad(mask, ((0, 0), (0, 64 - SEQ)))
    pos = pos_embed.reshape(SEQ, EMBED)
    out = _SC_KERNEL(tok2d, maskp, token_embed, pos)
    return out.reshape(BATCH, SEQ, EMBED)


# speedup vs baseline: 1.1737x; 1.0004x over previous
import functools

import jax
import jax.numpy as jnp
from jax import lax
from jax.experimental import pallas as pl
from jax.experimental.pallas import tpu as pltpu
from jax.experimental.pallas import tpu_sc as plsc

VOCAB = 1000000
EMBED = 64
SEQ = 50
BATCH = 4096

NC = 2
NS = 16
NW = NC * NS

TOKENS = BATCH * SEQ
PER_W = TOKENS // NW
SEQ_W = PER_W // SEQ
CHUNK = 400
SEQ_PER_CHUNK = CHUNK // SEQ
N_CHUNKS = PER_W // CHUNK
SUBG = 100
N_SUBG = CHUNK // SUBG
IDX_ROWS = PER_W // SUBG
RING = 4
N_OUTER = N_CHUNKS // RING


def _make_sc_kernel():
    mesh = plsc.VectorSubcoreMesh(core_axis_name="c", subcore_axis_name="s")

    @functools.partial(
        pl.kernel,
        mesh=mesh,
        out_type=jax.ShapeDtypeStruct((TOKENS, EMBED), jnp.float32),
        compiler_params=pltpu.CompilerParams(use_tc_tiling_on_sc=False),
        scratch_types=[
            pltpu.VMEM((IDX_ROWS, SUBG), jnp.int32),
            pltpu.VMEM((SEQ_W, 64), jnp.float32),
            pltpu.VMEM((RING, CHUNK, EMBED), jnp.float32),
            pltpu.VMEM((SEQ, EMBED), jnp.float32),
        ] + [pltpu.SemaphoreType.DMA] * (2 * RING),
    )
    def sc_kernel(tok_hbm, maskp_hbm, table_hbm, pos_hbm, out_hbm,
                  idx_v, maskp_v, rows_v, pos_v, *sems):
        gsem = sems[:RING]
        wsem = sems[RING:]
        wid = lax.axis_index("s") * NC + lax.axis_index("c")
        base = wid * PER_W

        pltpu.sync_copy(pos_hbm, pos_v)
        pltpu.sync_copy(
            tok_hbm.at[pl.ds(pl.multiple_of(wid * IDX_ROWS, IDX_ROWS),
                             IDX_ROWS)], idx_v)
        pltpu.sync_copy(
            maskp_hbm.at[pl.ds(pl.multiple_of(wid * SEQ_W, SEQ_W), SEQ_W)],
            maskp_v)

        lanes = lax.broadcasted_iota(jnp.int32, (16,), 0)
        dnums = lax.GatherDimensionNumbers(
            offset_dims=(), collapsed_slice_dims=(0,), start_index_map=(0,))

        def cbase_of(ci):
            return pl.multiple_of(base + ci * CHUNK, CHUNK)

        def fire_gather(b, ci):
            for g in range(N_SUBG):
                pltpu.async_copy(
                    table_hbm.at[idx_v.at[ci * N_SUBG + g]],
                    rows_v.at[b, pl.ds(g * SUBG, SUBG)],
                    gsem[b])

        def wait_gather(b, ci):
            for g in range(N_SUBG):
                pltpu.make_async_copy(
                    table_hbm.at[idx_v.at[ci * N_SUBG + g]],
                    rows_v.at[b, pl.ds(g * SUBG, SUBG)],
                    gsem[b]).wait()

        def fire_wb(b, ci):
            pltpu.async_copy(
                rows_v.at[b], out_hbm.at[pl.ds(cbase_of(ci), CHUNK)], wsem[b])

        def wait_wb(b):
            pltpu.make_async_copy(
                rows_v.at[b], out_hbm.at[pl.ds(cbase_of(0), CHUNK)],
                wsem[b]).wait()

        def compute(b, ci):
            srow0 = ci * SEQ_PER_CHUNK

            def s_body(s, c2):
                k16 = (s // 16) * 16
                lane = s % 16
                gidx = lanes * 0 + lane
                prow = [pos_v[s, pl.ds(dg * 16, 16)] for dg in range(4)]
                for si in range(SEQ_PER_CHUNK):
                    t = si * SEQ + s
                    mrow = maskp_v[srow0 + si, pl.ds(k16, 16)]
                    m = lax.gather(
                        mrow, gidx[:, None], dnums, (1,),
                        mode=lax.GatherScatterMode.PROMISE_IN_BOUNDS)
                    for dg in range(EMBED // 16):
                        sl = pl.ds(dg * 16, 16)
                        rows_v[b, t, sl] = (rows_v[b, t, sl]
                                            + prow[dg]) * m
                return c2

            lax.fori_loop(0, SEQ, s_body, 0)

        fire_gather(0, 0)
        fire_gather(1, 1)

        def outer_body(p, carry):
            for j in range(RING):
                ci = p * RING + j
                fb = (j + 2) % RING
                fci = ci + 2

                @pl.when(jnp.logical_and(fci >= RING, fci < N_CHUNKS))
                def _():
                    wait_wb(fb)

                @pl.when(fci < N_CHUNKS)
                def _():
                    fire_gather(fb, fci)

                wait_gather(j, ci)
                compute(j, ci)
                fire_wb(j, ci)
            return carry

        lax.fori_loop(0, N_OUTER, outer_body, 0)
        for b in range(RING):
            wait_wb(b)

    return sc_kernel


_SC_KERNEL = _make_sc_kernel()


@jax.jit
def kernel(tokens, mask, token_embed, pos_embed):
    tok2d = tokens.reshape(TOKENS // SUBG, SUBG).astype(jnp.int32)
    maskp = jnp.pad(mask, ((0, 0), (0, 64 - SEQ)))
    pos = pos_embed.reshape(SEQ, EMBED)
    out = _SC_KERNEL(tok2d, maskp, token_embed, pos)
    return out.reshape(BATCH, SEQ, EMBED)


# speedup vs baseline: 1.1748x; 1.0009x over previous
import functools

import jax
import jax.numpy as jnp
from jax import lax
from jax.experimental import pallas as pl
from jax.experimental.pallas import tpu as pltpu
from jax.experimental.pallas import tpu_sc as plsc

VOCAB = 1000000
EMBED = 64
SEQ = 50
BATCH = 4096

NC = 2
NS = 16
NW = NC * NS

TOKENS = BATCH * SEQ
PER_W = TOKENS // NW
SEQ_W = PER_W // SEQ
CHUNK = 200
SEQ_PER_CHUNK = CHUNK // SEQ
N_CHUNKS = PER_W // CHUNK
SUBG = 100
N_SUBG = CHUNK // SUBG
IDX_ROWS = PER_W // SUBG
RING = 8
LOOK = 5
N_OUTER = N_CHUNKS // RING


def _make_sc_kernel():
    mesh = plsc.VectorSubcoreMesh(core_axis_name="c", subcore_axis_name="s")

    @functools.partial(
        pl.kernel,
        mesh=mesh,
        out_type=jax.ShapeDtypeStruct((TOKENS, EMBED), jnp.float32),
        compiler_params=pltpu.CompilerParams(use_tc_tiling_on_sc=False),
        scratch_types=[
            pltpu.VMEM((IDX_ROWS, SUBG), jnp.int32),
            pltpu.VMEM((SEQ_W, 64), jnp.float32),
            pltpu.VMEM((RING, CHUNK, EMBED), jnp.float32),
            pltpu.VMEM((SEQ, EMBED), jnp.float32),
        ] + [pltpu.SemaphoreType.DMA] * (2 * RING),
    )
    def sc_kernel(tok_hbm, maskp_hbm, table_hbm, pos_hbm, out_hbm,
                  idx_v, maskp_v, rows_v, pos_v, *sems):
        gsem = sems[:RING]
        wsem = sems[RING:]
        wid = lax.axis_index("s") * NC + lax.axis_index("c")
        base = wid * PER_W

        pltpu.sync_copy(pos_hbm, pos_v)
        pltpu.sync_copy(
            tok_hbm.at[pl.ds(pl.multiple_of(wid * IDX_ROWS, IDX_ROWS),
                             IDX_ROWS)], idx_v)
        pltpu.sync_copy(
            maskp_hbm.at[pl.ds(pl.multiple_of(wid * SEQ_W, SEQ_W), SEQ_W)],
            maskp_v)

        lanes = lax.broadcasted_iota(jnp.int32, (16,), 0)
        dnums = lax.GatherDimensionNumbers(
            offset_dims=(), collapsed_slice_dims=(0,), start_index_map=(0,))

        def cbase_of(ci):
            return pl.multiple_of(base + ci * CHUNK, CHUNK)

        def fire_gather(b, ci):
            for g in range(N_SUBG):
                pltpu.async_copy(
                    table_hbm.at[idx_v.at[ci * N_SUBG + g]],
                    rows_v.at[b, pl.ds(g * SUBG, SUBG)],
                    gsem[b])

        def wait_gather(b, ci):
            for g in range(N_SUBG):
                pltpu.make_async_copy(
                    table_hbm.at[idx_v.at[ci * N_SUBG + g]],
                    rows_v.at[b, pl.ds(g * SUBG, SUBG)],
                    gsem[b]).wait()

        def fire_wb(b, ci):
            pltpu.async_copy(
                rows_v.at[b], out_hbm.at[pl.ds(cbase_of(ci), CHUNK)], wsem[b])

        def wait_wb(b):
            pltpu.make_async_copy(
                rows_v.at[b], out_hbm.at[pl.ds(cbase_of(0), CHUNK)],
                wsem[b]).wait()

        def compute(b, ci):
            srow0 = ci * SEQ_PER_CHUNK

            def s_body(s, c2):
                k16 = (s // 16) * 16
                lane = s % 16
                gidx = lanes * 0 + lane
                prow = [pos_v[s, pl.ds(dg * 16, 16)] for dg in range(4)]
                for si in range(SEQ_PER_CHUNK):
                    t = si * SEQ + s
                    mrow = maskp_v[srow0 + si, pl.ds(k16, 16)]
                    m = lax.gather(
                        mrow, gidx[:, None], dnums, (1,),
                        mode=lax.GatherScatterMode.PROMISE_IN_BOUNDS)
                    for dg in range(EMBED // 16):
                        sl = pl.ds(dg * 16, 16)
                        rows_v[b, t, sl] = (rows_v[b, t, sl]
                                            + prow[dg]) * m
                return c2

            lax.fori_loop(0, SEQ, s_body, 0)

        for b0 in range(LOOK):
            fire_gather(b0, b0)

        def outer_body(p, carry):
            for j in range(RING):
                ci = p * RING + j
                fb = (j + LOOK) % RING
                fci = ci + LOOK

                @pl.when(jnp.logical_and(fci >= RING, fci < N_CHUNKS))
                def _():
                    wait_wb(fb)

                @pl.when(fci < N_CHUNKS)
                def _():
                    fire_gather(fb, fci)

                wait_gather(j, ci)
                compute(j, ci)
                fire_wb(j, ci)
            return carry

        lax.fori_loop(0, N_OUTER, outer_body, 0)
        for b in range(RING):
            wait_wb(b)

    return sc_kernel


_SC_KERNEL = _make_sc_kernel()


@jax.jit
def kernel(tokens, mask, token_embed, pos_embed):
    tok2d = tokens.reshape(TOKENS // SUBG, SUBG).astype(jnp.int32)
    maskp = jnp.pad(mask, ((0, 0), (0, 64 - SEQ)))
    pos = pos_embed.reshape(SEQ, EMBED)
    out = _SC_KERNEL(tok2d, maskp, token_embed, pos)
    return out.reshape(BATCH, SEQ, EMBED)
